# Initial kernel scaffold; baseline (speedup 1.0000x reference)
import jax
import jax.numpy as jnp
from jax.experimental import pallas as pl


def kernel(x, edge_index, edge_attr, select_ca, params):
    raise NotImplementedError("write your pallas kernel here")


# speedup vs baseline: 2.3858x; 2.3858x over previous
import jax
import jax.numpy as jnp
from jax import lax
from jax.experimental import pallas as pl
from jax.experimental.pallas import tpu as pltpu
from jax.experimental.pallas import tpu_sc as plsc

N = 10000
E = 320000
C = 64
G = 128
IN_DIM = 4

_COMBOS = [(i, f, o) for i in range(3) for f in range(3)
           for o in range(abs(f - i), min(i + f + 1, 3))]

NC = 2
NS = 16
NW = NC * NS
CH = 128
EP = 323584
CPW = EP // (CH * NW)

ACC_N = 10112
RPS = ACC_N // NS

EB = 2048
NBLK = 1000


def _ssp(x):
    return jnp.maximum(x, 0.0) + jnp.log1p(jnp.exp(-jnp.abs(x))) - jnp.log(2.0)


def _elu(x):
    return jnp.where(x > 0, x, jnp.exp(jnp.minimum(x, 0.0)) - 1.0)


def _edgew_body(ea_ref, a_ref, amu_ref, w1_ref, b1_ref, bd_ref, b2_ref, w_ref):
    ea = ea_ref[...]
    r = jnp.sqrt(jnp.sum(ea * ea, axis=1, keepdims=True) + 1e-12)
    sr = r * a_ref[0, 0, 0]
    d = sr - amu_ref[0]
    rb = jnp.exp(-(d * d))
    a = jnp.maximum(
        jnp.dot(rb, w1_ref[0], preferred_element_type=jnp.float32)
        + b1_ref[0], 0.0)
    w = (jnp.dot(a, bd_ref[0], preferred_element_type=jnp.float32)
         + b2_ref[0])
    eb = pl.program_id(1)
    ids = eb * EB + lax.broadcasted_iota(jnp.int32, (EB, 1), 0)
    w_ref[0] = jnp.where(ids < E, w, 0.0)


def _edge_weights(ea_pad, aS, amuS, w1S, b1S, bdS, b2S):
    nblocks = EP // EB
    return pl.pallas_call(
        _edgew_body,
        grid=(10, nblocks),
        in_specs=[
            pl.BlockSpec((EB, 3), lambda g, eb: (eb, 0)),
            pl.BlockSpec((1, 1, 40), lambda g, eb: (g, 0, 0)),
            pl.BlockSpec((1, 1, 40), lambda g, eb: (g, 0, 0)),
            pl.BlockSpec((1, 40, 24), lambda g, eb: (g, 0, 0)),
            pl.BlockSpec((1, 1, 24), lambda g, eb: (g, 0, 0)),
            pl.BlockSpec((1, 24, G), lambda g, eb: (g, 0, 0)),
            pl.BlockSpec((1, 1, G), lambda g, eb: (g, 0, 0)),
        ],
        out_specs=pl.BlockSpec((1, EB, G), lambda g, eb: (g, eb, 0)),
        out_shape=jax.ShapeDtypeStruct((10, EP, G), jnp.float32),
    )(ea_pad, aS, amuS, w1S, b1S, bdS, b2S)


def _nodeh1_body(x_ref, lin1_ref, win_ref, h_ref):
    out = jnp.dot(x_ref[...], lin1_ref[...], preferred_element_type=jnp.float32)
    for g in range(2):
        h_ref[g] = jnp.dot(out, win_ref[g], preferred_element_type=jnp.float32)


def _node_h1(x, lin1, wincat):
    return pl.pallas_call(
        _nodeh1_body,
        grid=(N // NBLK,),
        in_specs=[
            pl.BlockSpec((NBLK, IN_DIM), lambda nb: (nb, 0)),
            pl.BlockSpec((IN_DIM, C), lambda nb: (0, 0)),
            pl.BlockSpec((2, C, G), lambda nb: (0, 0, 0)),
        ],
        out_specs=pl.BlockSpec((2, NBLK, G), lambda nb: (0, nb, 0)),
        out_shape=jax.ShapeDtypeStruct((2, N, G), jnp.float32),
    )(x, lin1, wincat)


def _make_sc_body(grp):
    def _sc_body(htab, wtab, src_hbm, dst_hbm, zero_hbm, out_hbm,
                 srcv, dstv, rows, wv, acc_sh, sem):
        c = lax.axis_index("c")
        s = lax.axis_index("s")
        wid = s * NC + c
        r0 = s * RPS

        def group_body(g, carry):
            pltpu.sync_copy(zero_hbm.at[pl.ds(r0, RPS)],
                            acc_sh.at[pl.ds(r0, RPS)])
            plsc.subcore_barrier()

            def chunk_body(i, carry2):
                base = (i * NW + wid) * CH
                pltpu.sync_copy(src_hbm.at[pl.ds(base, CH)], srcv)
                off = g * N
                for j in range(CH // 16):
                    sl = pl.ds(j * 16, 16)
                    srcv[sl] = srcv[sl] + off
                pltpu.sync_copy(dst_hbm.at[pl.ds(base, CH)], dstv)
                pltpu.async_copy(htab.at[srcv], rows, sem).wait()
                pltpu.sync_copy(wtab.at[pl.ds(g * EP + base, CH)], wv)

                def mul_body(k, carry3):
                    for cc in range(G // 16):
                        sl = pl.ds(cc * 16, 16)
                        rows[k, sl] = rows[k, sl] * wv[k, sl]
                    return carry3
                lax.fori_loop(0, CH, mul_body, 0, unroll=4)

                pltpu.sync_copy(rows, acc_sh.at[dstv], add=True)
                return carry2
            lax.fori_loop(0, CPW, chunk_body, 0)

            plsc.subcore_barrier()
            pltpu.sync_copy(acc_sh.at[pl.ds(r0, RPS)],
                            out_hbm.at[g].at[c].at[pl.ds(r0, RPS)])
            plsc.subcore_barrier()
            return carry
        lax.fori_loop(0, grp, group_body, 0)
    return _sc_body


def _sc_edge_pass(grp, htab, wtab, src, dst, zeros_tbl):
    mesh = plsc.VectorSubcoreMesh(core_axis_name="c", subcore_axis_name="s")
    fn = pl.kernel(
        _make_sc_body(grp),
        out_type=jax.ShapeDtypeStruct((grp, NC, ACC_N, G), jnp.float32),
        mesh=mesh,
        scratch_types=[
            pltpu.VMEM((CH,), jnp.int32),
            pltpu.VMEM((CH,), jnp.int32),
            pltpu.VMEM((CH, G), jnp.float32),
            pltpu.VMEM((CH, G), jnp.float32),
            pltpu.VMEM_SHARED((ACC_N, G), jnp.float32),
            pltpu.SemaphoreType.DMA,
        ],
    )
    return fn(htab, wtab, src, dst, zeros_tbl)


def _ec_body(agg_ref, wout_ref, lin2_ref, lin3_ref, win2_ref, h2_ref):
    outs = []
    for l in range(3):
        g, h = l // 2, l % 2
        a = (agg_ref[g, 0, :, h * C:(h + 1) * C]
             + agg_ref[g, 1, :, h * C:(h + 1) * C])
        o = jnp.dot(a, wout_ref[l], preferred_element_type=jnp.float32)
        nrm = jnp.sqrt(jnp.sum(o * o, axis=1, keepdims=True))
        o = o / (nrm + 1e-8)
        o = jnp.dot(o, lin2_ref[l], preferred_element_type=jnp.float32)
        o = _ssp(o)
        o = jnp.dot(o, lin3_ref[l], preferred_element_type=jnp.float32)
        outs.append(o)
    for k, (i, _f, _o) in enumerate(_COMBOS):
        g, h = k // 2, k % 2
        h2_ref[g, :, h * C:(h + 1) * C] = jnp.dot(
            outs[i], win2_ref[k], preferred_element_type=jnp.float32)
    h2_ref[7, :, C:2 * C] = jnp.zeros((NBLK, C), jnp.float32)


def _ec(agg1, woutS, lin2S, lin3S, win2S):
    return pl.pallas_call(
        _ec_body,
        grid=(N // NBLK,),
        in_specs=[
            pl.BlockSpec((2, NC, NBLK, G), lambda nb: (0, 0, nb, 0)),
            pl.BlockSpec((3, C, C), lambda nb: (0, 0, 0)),
            pl.BlockSpec((3, C, C), lambda nb: (0, 0, 0)),
            pl.BlockSpec((3, C, C), lambda nb: (0, 0, 0)),
            pl.BlockSpec((15, C, C), lambda nb: (0, 0, 0)),
        ],
        out_specs=pl.BlockSpec((8, NBLK, G), lambda nb: (0, nb, 0)),
        out_shape=jax.ShapeDtypeStruct((8, N, G), jnp.float32),
    )(agg1, woutS, lin2S, lin3S, win2S)


def _final_body(agg_ref, sel_ref,
                wout2_ref, lin40_ref, lin41_ref, lin42_ref,
                d1w_ref, d1b_ref, d2w_ref, d2b_ref, d3w_ref, d3b_ref,
                out_ref, acc_smem):
    lin4 = {0: lin40_ref, 1: lin41_ref, 2: lin42_ref}
    acc = {0: jnp.zeros((NBLK, C), jnp.float32),
           1: jnp.zeros((NBLK, C), jnp.float32),
           2: jnp.zeros((NBLK, C), jnp.float32)}
    pos = {0: 0, 1: 0, 2: 0}
    for k, (_i, _f, o) in enumerate(_COMBOS):
        g, h = k // 2, k % 2
        a = (agg_ref[g, 0, :, h * C:(h + 1) * C]
             + agg_ref[g, 1, :, h * C:(h + 1) * C])
        co = jnp.dot(a, wout2_ref[k], preferred_element_type=jnp.float32)
        j = pos[o]
        pos[o] += 1
        acc[o] = acc[o] + jnp.dot(co, lin4[o][j * C:(j + 1) * C, :],
                                  preferred_element_type=jnp.float32)
    feat = _ssp(acc[0]) + _ssp(acc[1]) + _ssp(acc[2])
    h = _elu(jnp.dot(feat, d1w_ref[...], preferred_element_type=jnp.float32)
             + d1b_ref[...])
    h = _elu(jnp.dot(h, d2w_ref[...], preferred_element_type=jnp.float32)
             + d2b_ref[...])
    pred = (jnp.dot(h, d3w_ref[...], preferred_element_type=jnp.float32)
            + d3b_ref[...])
    m = sel_ref[...] != 0
    psum = jnp.sum(jnp.where(m, pred, 0.0))
    pcnt = jnp.sum(m.astype(jnp.float32))

    @pl.when(pl.program_id(0) == 0)
    def _init():
        acc_smem[0] = 0.0
        acc_smem[1] = 0.0
    acc_smem[0] += psum
    acc_smem[1] += pcnt

    @pl.when(pl.program_id(0) == N // NBLK - 1)
    def _fin():
        out_ref[...] = jnp.full((1, 1), acc_smem[0] / acc_smem[1], jnp.float32)


def _final(agg2, sel2d, wout2S, lin40, lin41, lin42,
           d1w, d1b, d2w, d2b, d3w, d3b):
    full = lambda shape: pl.BlockSpec(shape, lambda nb: (0,) * len(shape))
    return pl.pallas_call(
        _final_body,
        grid=(N // NBLK,),
        in_specs=[
            pl.BlockSpec((8, NC, NBLK, G), lambda nb: (0, 0, nb, 0)),
            pl.BlockSpec((NBLK, 1), lambda nb: (nb, 0)),
            full((15, C, C)),
            full((3 * C, C)),
            full((6 * C, C)),
            full((6 * C, C)),
            full((C, 250)),
            full((1, 250)),
            full((250, 150)),
            full((1, 150)),
            full((150, 1)),
            full((1, 1)),
        ],
        out_specs=pl.BlockSpec((1, 1), lambda nb: (0, 0)),
        out_shape=jax.ShapeDtypeStruct((1, 1), jnp.float32),
        scratch_shapes=[pltpu.SMEM((2,), jnp.float32)],
    )(agg2, sel2d, wout2S, lin40, lin41, lin42,
      d1w, d1b, d2w, d2b, d3w, d3b)


def _bd2(ws):
    z = jnp.zeros((24, G), jnp.float32)
    for j, w in enumerate(ws):
        z = z.at[12 * j:12 * (j + 1), 64 * j:64 * (j + 1)].set(w)
    return z


def _group_stacks(params):
    aS, amuS, w1S, b1S, bdS, b2S = [], [], [], [], [], []

    def add_group(ps, max_radius, n_basis):
        mu = jnp.linspace(0.0, max_radius, n_basis)
        a = 1.0 / (mu[1] - mu[0])
        mu = jnp.pad(mu, (0, 40 - n_basis), constant_values=1e6)
        aS.append(jnp.full((40,), a, jnp.float32))
        amuS.append(a * mu)
        w1 = jnp.concatenate([p['w1'] for p in ps], axis=1)
        w1 = jnp.pad(w1, ((0, 40 - n_basis), (0, 24 - w1.shape[1])))
        w1S.append(w1)
        b1 = jnp.concatenate([p['b1'] for p in ps])
        b1S.append(jnp.pad(b1, (0, 24 - b1.shape[0])))
        bdS.append(_bd2([p['w2'] for p in ps]))
        b2 = jnp.concatenate([p['b2'] for p in ps])
        b2S.append(jnp.pad(b2, (0, G - b2.shape[0])))

    add_group([params['conv1_%d' % l] for l in range(2)], 10.0, 20)
    add_group([params['conv1_2']], 10.0, 20)
    for g in range(8):
        ps = [params['conv2_%d%d%d' % c] for c in _COMBOS[2 * g:2 * g + 2]]
        add_group(ps, 20.0, 40)

    return (jnp.stack(aS)[:, None, :], jnp.stack(amuS)[:, None, :],
            jnp.stack(w1S), jnp.stack(b1S)[:, None, :],
            jnp.stack(bdS), jnp.stack(b2S)[:, None, :])


def kernel(x, edge_index, edge_attr, select_ca, params):
    src = edge_index[0].astype(jnp.int32)
    dst = edge_index[1].astype(jnp.int32)
    pad = EP - E
    padidx = (jnp.arange(pad, dtype=jnp.int32) * 997) % N
    src_p = jnp.concatenate([src, padidx])
    dst_p = jnp.concatenate([dst, padidx])
    ea_p = jnp.concatenate(
        [edge_attr, jnp.zeros((pad, 3), jnp.float32)], axis=0)
    zeros_tbl = jnp.zeros((ACC_N, G), jnp.float32)

    aS, amuS, w1S, b1S, bdS, b2S = _group_stacks(params)
    wins1 = [params['conv1_%d' % l]['win'] for l in range(3)]
    wincat1 = jnp.stack([
        jnp.concatenate([wins1[0], wins1[1]], axis=1),
        jnp.concatenate([wins1[2], jnp.zeros((C, C), jnp.float32)], axis=1),
    ])
    woutS1 = jnp.stack([params['conv1_%d' % l]['wout'] for l in range(3)])
    lin2S = jnp.stack([params['lin2_%d' % l] for l in range(3)])
    lin3S = jnp.stack([params['lin3_%d' % l] for l in range(3)])
    win2S = jnp.stack([params['conv2_%d%d%d' % c]['win'] for c in _COMBOS])
    wout2S = jnp.stack([params['conv2_%d%d%d' % c]['wout'] for c in _COMBOS])

    W = _edge_weights(ea_p, aS, amuS, w1S, b1S, bdS, b2S)
    H1 = _node_h1(x, params['lin1'], wincat1)

    agg1 = _sc_edge_pass(2, H1.reshape(2 * N, G),
                         W[:2].reshape(2 * EP, G),
                         src_p, dst_p, zeros_tbl)
    H2 = _ec(agg1, woutS1, lin2S, lin3S, win2S)
    agg2 = _sc_edge_pass(8, H2.reshape(8 * N, G),
                         W[2:].reshape(8 * EP, G),
                         src_p, dst_p, zeros_tbl)

    sel2d = select_ca.reshape(N, 1).astype(jnp.int32)
    out = _final(agg2, sel2d, wout2S,
                 params['lin40'], params['lin41'], params['lin42'],
                 params['d1w'], params['d1b'].reshape(1, 250),
                 params['d2w'], params['d2b'].reshape(1, 150),
                 params['d3w'], params['d3b'].reshape(1, 1))
    return out[0, 0]


# speedup vs baseline: 3.2501x; 1.3623x over previous
import jax
import jax.numpy as jnp
from jax import lax
from jax.experimental import pallas as pl
from jax.experimental.pallas import tpu as pltpu
from jax.experimental.pallas import tpu_sc as plsc

N = 10000
E = 320000
C = 64
G = 128
IN_DIM = 4

_COMBOS = [(i, f, o) for i in range(3) for f in range(3)
           for o in range(abs(f - i), min(i + f + 1, 3))]

NC = 2
NS = 16
NW = NC * NS
CH = 64
EP = 331776
CPW = EP // (CH * NW)
NRING = 3

ACC_N = 10112
RPS = ACC_N // NS

EB = 2048
NBLK = 1000


def _ssp(x):
    return jnp.maximum(x, 0.0) + jnp.log1p(jnp.exp(-jnp.abs(x))) - jnp.log(2.0)


def _elu(x):
    return jnp.where(x > 0, x, jnp.exp(jnp.minimum(x, 0.0)) - 1.0)


def _edgew_body(ea_ref, a_ref, amu_ref, w1_ref, b1_ref, bd_ref, b2_ref, w_ref):
    ea = ea_ref[...]
    r = jnp.sqrt(jnp.sum(ea * ea, axis=1, keepdims=True) + 1e-12)
    sr = r * a_ref[0, 0, 0]
    d = sr - amu_ref[0]
    rb = jnp.exp(-(d * d))
    a = jnp.maximum(
        jnp.dot(rb, w1_ref[0], preferred_element_type=jnp.float32)
        + b1_ref[0], 0.0)
    w = (jnp.dot(a, bd_ref[0], preferred_element_type=jnp.float32)
         + b2_ref[0])
    eb = pl.program_id(1)
    ids = eb * EB + lax.broadcasted_iota(jnp.int32, (EB, 1), 0)
    w_ref[0] = jnp.where(ids < E, w, 0.0)


def _edge_weights(ea_pad, aS, amuS, w1S, b1S, bdS, b2S):
    nblocks = EP // EB
    return pl.pallas_call(
        _edgew_body,
        grid=(10, nblocks),
        in_specs=[
            pl.BlockSpec((EB, 3), lambda g, eb: (eb, 0)),
            pl.BlockSpec((1, 1, 40), lambda g, eb: (g, 0, 0)),
            pl.BlockSpec((1, 1, 40), lambda g, eb: (g, 0, 0)),
            pl.BlockSpec((1, 40, 24), lambda g, eb: (g, 0, 0)),
            pl.BlockSpec((1, 1, 24), lambda g, eb: (g, 0, 0)),
            pl.BlockSpec((1, 24, G), lambda g, eb: (g, 0, 0)),
            pl.BlockSpec((1, 1, G), lambda g, eb: (g, 0, 0)),
        ],
        out_specs=pl.BlockSpec((1, EB, G), lambda g, eb: (g, eb, 0)),
        out_shape=jax.ShapeDtypeStruct((10, EP, G), jnp.float32),
    )(ea_pad, aS, amuS, w1S, b1S, bdS, b2S)


def _nodeh1_body(x_ref, lin1_ref, win_ref, h_ref):
    out = jnp.dot(x_ref[...], lin1_ref[...], preferred_element_type=jnp.float32)
    for g in range(2):
        h_ref[g] = jnp.dot(out, win_ref[g], preferred_element_type=jnp.float32)


def _node_h1(x, lin1, wincat):
    return pl.pallas_call(
        _nodeh1_body,
        grid=(N // NBLK,),
        in_specs=[
            pl.BlockSpec((NBLK, IN_DIM), lambda nb: (nb, 0)),
            pl.BlockSpec((IN_DIM, C), lambda nb: (0, 0)),
            pl.BlockSpec((2, C, G), lambda nb: (0, 0, 0)),
        ],
        out_specs=pl.BlockSpec((2, NBLK, G), lambda nb: (0, nb, 0)),
        out_shape=jax.ShapeDtypeStruct((2, N, G), jnp.float32),
    )(x, lin1, wincat)


def _make_sc_body(grp):
    def _sc_body(htab, wtab, idxpk, zero_hbm, out_hbm,
                 idxb, rows, wv, acc_sh, gsem, wsem):
        c = lax.axis_index("c")
        s = lax.axis_index("s")
        wid = s * NC + c
        r0 = s * RPS

        def group_body(g, carry):
            pltpu.sync_copy(zero_hbm.at[pl.ds(r0, RPS)],
                            acc_sh.at[pl.ds(r0, RPS)])
            plsc.subcore_barrier()
            off = g * N

            def issue(ch, b):
                gc = wid * CPW + ch
                pltpu.sync_copy(idxpk.at[gc], idxb.at[b])
                for j in range(CH // 16):
                    sl = pl.ds(j * 16, 16)
                    idxb[b, 0, sl] = idxb[b, 0, sl] + off
                pltpu.async_copy(htab.at[idxb.at[b, 0]], rows.at[b],
                                 gsem.at[b])
                pltpu.async_copy(wtab.at[pl.ds(g * EP + gc * CH, CH)],
                                 wv.at[b], wsem.at[b])

            issue(0, 0)
            issue(1, 1)

            def outer(io, carry2):
                for b in range(NRING):
                    ch = io * NRING + b
                    gc = wid * CPW + ch
                    pltpu.make_async_copy(htab.at[idxb.at[b, 0]],
                                          rows.at[b], gsem.at[b]).wait()
                    pltpu.make_async_copy(wtab.at[pl.ds(g * EP + gc * CH, CH)],
                                          wv.at[b], wsem.at[b]).wait()

                    rb = rows.at[b]
                    wb = wv.at[b]

                    def mul_body(k, carry3):
                        for cc in range(G // 16):
                            sl = pl.ds(cc * 16, 16)
                            rb[k, sl] = rb[k, sl] * wb[k, sl]
                        return carry3
                    lax.fori_loop(0, CH, mul_body, 0, unroll=4)

                    pltpu.sync_copy(rows.at[b], acc_sh.at[idxb.at[b, 1]],
                                    add=True)

                    @pl.when(ch + 2 < CPW)
                    def _issue_ahead():
                        issue(ch + 2, (b + 2) % NRING)
                return carry2
            lax.fori_loop(0, CPW // NRING, outer, 0)

            plsc.subcore_barrier()
            pltpu.sync_copy(acc_sh.at[pl.ds(r0, RPS)],
                            out_hbm.at[g].at[c].at[pl.ds(r0, RPS)])
            plsc.subcore_barrier()
            return carry
        lax.fori_loop(0, grp, group_body, 0)
    return _sc_body


def _sc_edge_pass(grp, htab, wtab, idxpk, zeros_tbl):
    mesh = plsc.VectorSubcoreMesh(core_axis_name="c", subcore_axis_name="s")
    fn = pl.kernel(
        _make_sc_body(grp),
        out_type=jax.ShapeDtypeStruct((grp, NC, ACC_N, G), jnp.float32),
        mesh=mesh,
        scratch_types=[
            pltpu.VMEM((NRING, 2, CH), jnp.int32),
            pltpu.VMEM((NRING, CH, G), jnp.float32),
            pltpu.VMEM((NRING, CH, G), jnp.float32),
            pltpu.VMEM_SHARED((ACC_N, G), jnp.float32),
            pltpu.SemaphoreType.DMA((NRING,)),
            pltpu.SemaphoreType.DMA((NRING,)),
        ],
    )
    return fn(htab, wtab, idxpk, zeros_tbl)


def _ec_body(agg_ref, wout_ref, lin2_ref, lin3_ref, win2_ref, h2_ref):
    outs = []
    for l in range(3):
        g, h = l // 2, l % 2
        a = (agg_ref[g, 0, :, h * C:(h + 1) * C]
             + agg_ref[g, 1, :, h * C:(h + 1) * C])
        o = jnp.dot(a, wout_ref[l], preferred_element_type=jnp.float32)
        nrm = jnp.sqrt(jnp.sum(o * o, axis=1, keepdims=True))
        o = o / (nrm + 1e-8)
        o = jnp.dot(o, lin2_ref[l], preferred_element_type=jnp.float32)
        o = _ssp(o)
        o = jnp.dot(o, lin3_ref[l], preferred_element_type=jnp.float32)
        outs.append(o)
    for k, (i, _f, _o) in enumerate(_COMBOS):
        g, h = k // 2, k % 2
        h2_ref[g, :, h * C:(h + 1) * C] = jnp.dot(
            outs[i], win2_ref[k], preferred_element_type=jnp.float32)
    h2_ref[7, :, C:2 * C] = jnp.zeros((NBLK, C), jnp.float32)


def _ec(agg1, woutS, lin2S, lin3S, win2S):
    return pl.pallas_call(
        _ec_body,
        grid=(N // NBLK,),
        in_specs=[
            pl.BlockSpec((2, NC, NBLK, G), lambda nb: (0, 0, nb, 0)),
            pl.BlockSpec((3, C, C), lambda nb: (0, 0, 0)),
            pl.BlockSpec((3, C, C), lambda nb: (0, 0, 0)),
            pl.BlockSpec((3, C, C), lambda nb: (0, 0, 0)),
            pl.BlockSpec((15, C, C), lambda nb: (0, 0, 0)),
        ],
        out_specs=pl.BlockSpec((8, NBLK, G), lambda nb: (0, nb, 0)),
        out_shape=jax.ShapeDtypeStruct((8, N, G), jnp.float32),
    )(agg1, woutS, lin2S, lin3S, win2S)


def _final_body(agg_ref, sel_ref,
                wout2_ref, lin40_ref, lin41_ref, lin42_ref,
                d1w_ref, d1b_ref, d2w_ref, d2b_ref, d3w_ref, d3b_ref,
                out_ref, acc_smem):
    lin4 = {0: lin40_ref, 1: lin41_ref, 2: lin42_ref}
    acc = {0: jnp.zeros((NBLK, C), jnp.float32),
           1: jnp.zeros((NBLK, C), jnp.float32),
           2: jnp.zeros((NBLK, C), jnp.float32)}
    pos = {0: 0, 1: 0, 2: 0}
    for k, (_i, _f, o) in enumerate(_COMBOS):
        g, h = k // 2, k % 2
        a = (agg_ref[g, 0, :, h * C:(h + 1) * C]
             + agg_ref[g, 1, :, h * C:(h + 1) * C])
        co = jnp.dot(a, wout2_ref[k], preferred_element_type=jnp.float32)
        j = pos[o]
        pos[o] += 1
        acc[o] = acc[o] + jnp.dot(co, lin4[o][j * C:(j + 1) * C, :],
                                  preferred_element_type=jnp.float32)
    feat = _ssp(acc[0]) + _ssp(acc[1]) + _ssp(acc[2])
    h = _elu(jnp.dot(feat, d1w_ref[...], preferred_element_type=jnp.float32)
             + d1b_ref[...])
    h = _elu(jnp.dot(h, d2w_ref[...], preferred_element_type=jnp.float32)
             + d2b_ref[...])
    pred = (jnp.dot(h, d3w_ref[...], preferred_element_type=jnp.float32)
            + d3b_ref[...])
    m = sel_ref[...] != 0
    psum = jnp.sum(jnp.where(m, pred, 0.0))
    pcnt = jnp.sum(m.astype(jnp.float32))

    @pl.when(pl.program_id(0) == 0)
    def _init():
        acc_smem[0] = 0.0
        acc_smem[1] = 0.0
    acc_smem[0] += psum
    acc_smem[1] += pcnt

    @pl.when(pl.program_id(0) == N // NBLK - 1)
    def _fin():
        out_ref[...] = jnp.full((1, 1), acc_smem[0] / acc_smem[1], jnp.float32)


def _final(agg2, sel2d, wout2S, lin40, lin41, lin42,
           d1w, d1b, d2w, d2b, d3w, d3b):
    full = lambda shape: pl.BlockSpec(shape, lambda nb: (0,) * len(shape))
    return pl.pallas_call(
        _final_body,
        grid=(N // NBLK,),
        in_specs=[
            pl.BlockSpec((8, NC, NBLK, G), lambda nb: (0, 0, nb, 0)),
            pl.BlockSpec((NBLK, 1), lambda nb: (nb, 0)),
            full((15, C, C)),
            full((3 * C, C)),
            full((6 * C, C)),
            full((6 * C, C)),
            full((C, 250)),
            full((1, 250)),
            full((250, 150)),
            full((1, 150)),
            full((150, 1)),
            full((1, 1)),
        ],
        out_specs=pl.BlockSpec((1, 1), lambda nb: (0, 0)),
        out_shape=jax.ShapeDtypeStruct((1, 1), jnp.float32),
        scratch_shapes=[pltpu.SMEM((2,), jnp.float32)],
    )(agg2, sel2d, wout2S, lin40, lin41, lin42,
      d1w, d1b, d2w, d2b, d3w, d3b)


def _bd2(ws):
    z = jnp.zeros((24, G), jnp.float32)
    for j, w in enumerate(ws):
        z = z.at[12 * j:12 * (j + 1), 64 * j:64 * (j + 1)].set(w)
    return z


def _group_stacks(params):
    aS, amuS, w1S, b1S, bdS, b2S = [], [], [], [], [], []

    def add_group(ps, max_radius, n_basis):
        mu = jnp.linspace(0.0, max_radius, n_basis)
        a = 1.0 / (mu[1] - mu[0])
        mu = jnp.pad(mu, (0, 40 - n_basis), constant_values=1e6)
        aS.append(jnp.full((40,), a, jnp.float32))
        amuS.append(a * mu)
        w1 = jnp.concatenate([p['w1'] for p in ps], axis=1)
        w1 = jnp.pad(w1, ((0, 40 - n_basis), (0, 24 - w1.shape[1])))
        w1S.append(w1)
        b1 = jnp.concatenate([p['b1'] for p in ps])
        b1S.append(jnp.pad(b1, (0, 24 - b1.shape[0])))
        bdS.append(_bd2([p['w2'] for p in ps]))
        b2 = jnp.concatenate([p['b2'] for p in ps])
        b2S.append(jnp.pad(b2, (0, G - b2.shape[0])))

    add_group([params['conv1_%d' % l] for l in range(2)], 10.0, 20)
    add_group([params['conv1_2']], 10.0, 20)
    for g in range(8):
        ps = [params['conv2_%d%d%d' % c] for c in _COMBOS[2 * g:2 * g + 2]]
        add_group(ps, 20.0, 40)

    return (jnp.stack(aS)[:, None, :], jnp.stack(amuS)[:, None, :],
            jnp.stack(w1S), jnp.stack(b1S)[:, None, :],
            jnp.stack(bdS), jnp.stack(b2S)[:, None, :])


def kernel(x, edge_index, edge_attr, select_ca, params):
    src = edge_index[0].astype(jnp.int32)
    dst = edge_index[1].astype(jnp.int32)
    pad = EP - E
    padidx = (jnp.arange(pad, dtype=jnp.int32) * 997) % N
    src_p = jnp.concatenate([src, padidx])
    dst_p = jnp.concatenate([dst, padidx])
    idxpk = jnp.stack([src_p.reshape(EP // CH, CH),
                       dst_p.reshape(EP // CH, CH)], axis=1)
    ea_p = jnp.concatenate(
        [edge_attr, jnp.zeros((pad, 3), jnp.float32)], axis=0)
    zeros_tbl = jnp.zeros((ACC_N, G), jnp.float32)

    aS, amuS, w1S, b1S, bdS, b2S = _group_stacks(params)
    wins1 = [params['conv1_%d' % l]['win'] for l in range(3)]
    wincat1 = jnp.stack([
        jnp.concatenate([wins1[0], wins1[1]], axis=1),
        jnp.concatenate([wins1[2], jnp.zeros((C, C), jnp.float32)], axis=1),
    ])
    woutS1 = jnp.stack([params['conv1_%d' % l]['wout'] for l in range(3)])
    lin2S = jnp.stack([params['lin2_%d' % l] for l in range(3)])
    lin3S = jnp.stack([params['lin3_%d' % l] for l in range(3)])
    win2S = jnp.stack([params['conv2_%d%d%d' % c]['win'] for c in _COMBOS])
    wout2S = jnp.stack([params['conv2_%d%d%d' % c]['wout'] for c in _COMBOS])

    W = _edge_weights(ea_p, aS, amuS, w1S, b1S, bdS, b2S)
    H1 = _node_h1(x, params['lin1'], wincat1)

    agg1 = _sc_edge_pass(2, H1.reshape(2 * N, G),
                         W[:2].reshape(2 * EP, G),
                         idxpk, zeros_tbl)
    H2 = _ec(agg1, woutS1, lin2S, lin3S, win2S)
    agg2 = _sc_edge_pass(8, H2.reshape(8 * N, G),
                         W[2:].reshape(8 * EP, G),
                         idxpk, zeros_tbl)

    sel2d = select_ca.reshape(N, 1).astype(jnp.int32)
    out = _final(agg2, sel2d, wout2S,
                 params['lin40'], params['lin41'], params['lin42'],
                 params['d1w'], params['d1b'].reshape(1, 250),
                 params['d2w'], params['d2b'].reshape(1, 150),
                 params['d3w'], params['d3b'].reshape(1, 1))
    return out[0, 0]


# speedup vs baseline: 5.2016x; 1.6004x over previous
import jax
import jax.numpy as jnp
from jax import lax
from jax.experimental import pallas as pl
from jax.experimental.pallas import tpu as pltpu
from jax.experimental.pallas import tpu_sc as plsc

N = 10000
E = 320000
C = 64
G = 128
IN_DIM = 4

_COMBOS = [(i, f, o) for i in range(3) for f in range(3)
           for o in range(abs(f - i), min(i + f + 1, 3))]

NC = 2
NS = 16
NW = NC * NS
CH = 64
EP = 331776
CPW = EP // (CH * NW)
NRING = 3

ACC_N = 10112
RPS = ACC_N // NS

EB = 2048
NBLK = 1000


def _ssp(x):
    return jnp.maximum(x, 0.0) + jnp.log1p(jnp.exp(-jnp.abs(x))) - jnp.log(2.0)


def _elu(x):
    return jnp.where(x > 0, x, jnp.exp(jnp.minimum(x, 0.0)) - 1.0)


def _make_edgew_body(ng):
    def _edgew_body(ea_ref, a_ref, amu_ref, w1_ref, b1_ref, bd_ref, b2_ref,
                    w_ref):
        ea = ea_ref[...]
        r = jnp.sqrt(jnp.sum(ea * ea, axis=1, keepdims=True) + 1e-12)
        sr = r * a_ref[0, 0]
        d = sr - amu_ref[...]
        rb = jnp.exp(-(d * d))
        a = jnp.maximum(
            jnp.dot(rb, w1_ref[...], preferred_element_type=jnp.float32)
            + b1_ref[...], 0.0)
        eb = pl.program_id(0)
        ids = eb * EB + lax.broadcasted_iota(jnp.int32, (EB, 1), 0)
        valid = ids < E
        for g in range(ng):
            w = (jnp.dot(a[:, 24 * g:24 * (g + 1)], bd_ref[g],
                         preferred_element_type=jnp.float32) + b2_ref[g])
            w_ref[g] = jnp.where(valid, w, 0.0)
    return _edgew_body


def _edge_weights(ng, nb, ea_pad, aS, amuS, w1S, b1S, bdS, b2S):
    nblocks = EP // EB
    return pl.pallas_call(
        _make_edgew_body(ng),
        grid=(nblocks,),
        in_specs=[
            pl.BlockSpec((EB, 3), lambda eb: (eb, 0)),
            pl.BlockSpec((1, 1), lambda eb: (0, 0)),
            pl.BlockSpec((1, nb), lambda eb: (0, 0)),
            pl.BlockSpec((nb, 24 * ng), lambda eb: (0, 0)),
            pl.BlockSpec((1, 24 * ng), lambda eb: (0, 0)),
            pl.BlockSpec((ng, 24, G), lambda eb: (0, 0, 0)),
            pl.BlockSpec((ng, 1, G), lambda eb: (0, 0, 0)),
        ],
        out_specs=pl.BlockSpec((ng, EB, G), lambda eb: (0, eb, 0)),
        out_shape=jax.ShapeDtypeStruct((ng, EP, G), jnp.float32),
    )(ea_pad, aS, amuS, w1S, b1S, bdS, b2S)


def _nodeh1_body(x_ref, lin1_ref, win_ref, h_ref):
    out = jnp.dot(x_ref[...], lin1_ref[...], preferred_element_type=jnp.float32)
    for g in range(2):
        h_ref[g] = jnp.dot(out, win_ref[g], preferred_element_type=jnp.float32)


def _node_h1(x, lin1, wincat):
    return pl.pallas_call(
        _nodeh1_body,
        grid=(N // NBLK,),
        in_specs=[
            pl.BlockSpec((NBLK, IN_DIM), lambda nb: (nb, 0)),
            pl.BlockSpec((IN_DIM, C), lambda nb: (0, 0)),
            pl.BlockSpec((2, C, G), lambda nb: (0, 0, 0)),
        ],
        out_specs=pl.BlockSpec((2, NBLK, G), lambda nb: (0, nb, 0)),
        out_shape=jax.ShapeDtypeStruct((2, N, G), jnp.float32),
    )(x, lin1, wincat)


def _make_sc_body(grp):
    def _sc_body(htab, wtab, idxpk, zero_hbm, out_hbm,
                 idxb, rows, wv, acc_sh, gsem, wsem, ssem):
        c = lax.axis_index("c")
        s = lax.axis_index("s")
        wid = s * NC + c
        r0 = s * RPS

        def group_body(g, carry):
            pltpu.sync_copy(zero_hbm.at[pl.ds(r0, RPS)],
                            acc_sh.at[pl.ds(r0, RPS)])
            plsc.subcore_barrier()
            off = g * N

            def issue(ch, b):
                gc = wid * CPW + ch
                pltpu.sync_copy(idxpk.at[gc], idxb.at[b])
                for j in range(CH // 16):
                    sl = pl.ds(j * 16, 16)
                    idxb[b, 0, sl] = idxb[b, 0, sl] + off
                pltpu.async_copy(htab.at[idxb.at[b, 0]], rows.at[b],
                                 gsem.at[b])
                pltpu.async_copy(wtab.at[pl.ds(g * EP + gc * CH, CH)],
                                 wv.at[b], wsem.at[b])

            issue(0, 0)
            issue(1, 1)

            def outer(io, carry2):
                for b in range(NRING):
                    ch = io * NRING + b
                    gc = wid * CPW + ch
                    pltpu.make_async_copy(htab.at[idxb.at[b, 0]],
                                          rows.at[b], gsem.at[b]).wait()
                    pltpu.make_async_copy(wtab.at[pl.ds(g * EP + gc * CH, CH)],
                                          wv.at[b], wsem.at[b]).wait()

                    rb = rows.at[b]
                    wb = wv.at[b]

                    def mul_body(k, carry3):
                        for cc in range(G // 16):
                            sl = pl.ds(cc * 16, 16)
                            rb[k, sl] = rb[k, sl] * wb[k, sl]
                        return carry3
                    lax.fori_loop(0, CH, mul_body, 0, unroll=4)

                    pltpu.async_copy(rows.at[b], acc_sh.at[idxb.at[b, 1]],
                                     ssem.at[b], add=True)

                    @pl.when(ch + 2 < CPW)
                    def _issue_ahead():
                        b2 = (b + 2) % NRING

                        @pl.when(ch >= 1)
                        def _wait_prev_scatter():
                            pltpu.make_async_copy(
                                rows.at[b2], acc_sh.at[idxb.at[b2, 1]],
                                ssem.at[b2]).wait()
                        issue(ch + 2, b2)
                return carry2
            lax.fori_loop(0, CPW // NRING, outer, 0)

            for bl in range(NRING):
                pltpu.make_async_copy(rows.at[bl], acc_sh.at[idxb.at[bl, 1]],
                                      ssem.at[bl]).wait()
            plsc.subcore_barrier()
            pltpu.sync_copy(acc_sh.at[pl.ds(r0, RPS)],
                            out_hbm.at[g].at[c].at[pl.ds(r0, RPS)])
            plsc.subcore_barrier()
            return carry
        lax.fori_loop(0, grp, group_body, 0)
    return _sc_body


def _sc_edge_pass(grp, htab, wtab, idxpk, zeros_tbl):
    mesh = plsc.VectorSubcoreMesh(core_axis_name="c", subcore_axis_name="s")
    fn = pl.kernel(
        _make_sc_body(grp),
        out_type=jax.ShapeDtypeStruct((grp, NC, ACC_N, G), jnp.float32),
        mesh=mesh,
        scratch_types=[
            pltpu.VMEM((NRING, 2, CH), jnp.int32),
            pltpu.VMEM((NRING, CH, G), jnp.float32),
            pltpu.VMEM((NRING, CH, G), jnp.float32),
            pltpu.VMEM_SHARED((ACC_N, G), jnp.float32),
            pltpu.SemaphoreType.DMA((NRING,)),
            pltpu.SemaphoreType.DMA((NRING,)),
            pltpu.SemaphoreType.DMA((NRING,)),
        ],
    )
    return fn(htab, wtab, idxpk, zeros_tbl)


def _ec_body(agg_ref, wout_ref, lin2_ref, lin3_ref, win2_ref, h2_ref):
    outs = []
    for l in range(3):
        g, h = l // 2, l % 2
        a = (agg_ref[g, 0, :, h * C:(h + 1) * C]
             + agg_ref[g, 1, :, h * C:(h + 1) * C])
        o = jnp.dot(a, wout_ref[l], preferred_element_type=jnp.float32)
        nrm = jnp.sqrt(jnp.sum(o * o, axis=1, keepdims=True))
        o = o / (nrm + 1e-8)
        o = jnp.dot(o, lin2_ref[l], preferred_element_type=jnp.float32)
        o = _ssp(o)
        o = jnp.dot(o, lin3_ref[l], preferred_element_type=jnp.float32)
        outs.append(o)
    for k, (i, _f, _o) in enumerate(_COMBOS):
        g, h = k // 2, k % 2
        h2_ref[g, :, h * C:(h + 1) * C] = jnp.dot(
            outs[i], win2_ref[k], preferred_element_type=jnp.float32)
    h2_ref[7, :, C:2 * C] = jnp.zeros((NBLK, C), jnp.float32)


def _ec(agg1, woutS, lin2S, lin3S, win2S):
    return pl.pallas_call(
        _ec_body,
        grid=(N // NBLK,),
        in_specs=[
            pl.BlockSpec((2, NC, NBLK, G), lambda nb: (0, 0, nb, 0)),
            pl.BlockSpec((3, C, C), lambda nb: (0, 0, 0)),
            pl.BlockSpec((3, C, C), lambda nb: (0, 0, 0)),
            pl.BlockSpec((3, C, C), lambda nb: (0, 0, 0)),
            pl.BlockSpec((15, C, C), lambda nb: (0, 0, 0)),
        ],
        out_specs=pl.BlockSpec((8, NBLK, G), lambda nb: (0, nb, 0)),
        out_shape=jax.ShapeDtypeStruct((8, N, G), jnp.float32),
    )(agg1, woutS, lin2S, lin3S, win2S)


def _final_body(agg_ref, sel_ref,
                wout2_ref, lin40_ref, lin41_ref, lin42_ref,
                d1w_ref, d1b_ref, d2w_ref, d2b_ref, d3w_ref, d3b_ref,
                out_ref, acc_smem):
    lin4 = {0: lin40_ref, 1: lin41_ref, 2: lin42_ref}
    acc = {0: jnp.zeros((NBLK, C), jnp.float32),
           1: jnp.zeros((NBLK, C), jnp.float32),
           2: jnp.zeros((NBLK, C), jnp.float32)}
    pos = {0: 0, 1: 0, 2: 0}
    for k, (_i, _f, o) in enumerate(_COMBOS):
        g, h = k // 2, k % 2
        a = (agg_ref[g, 0, :, h * C:(h + 1) * C]
             + agg_ref[g, 1, :, h * C:(h + 1) * C])
        co = jnp.dot(a, wout2_ref[k], preferred_element_type=jnp.float32)
        j = pos[o]
        pos[o] += 1
        acc[o] = acc[o] + jnp.dot(co, lin4[o][j * C:(j + 1) * C, :],
                                  preferred_element_type=jnp.float32)
    feat = _ssp(acc[0]) + _ssp(acc[1]) + _ssp(acc[2])
    h = _elu(jnp.dot(feat, d1w_ref[...], preferred_element_type=jnp.float32)
             + d1b_ref[...])
    h = _elu(jnp.dot(h, d2w_ref[...], preferred_element_type=jnp.float32)
             + d2b_ref[...])
    pred = (jnp.dot(h, d3w_ref[...], preferred_element_type=jnp.float32)
            + d3b_ref[...])
    m = sel_ref[...] != 0
    psum = jnp.sum(jnp.where(m, pred, 0.0))
    pcnt = jnp.sum(m.astype(jnp.float32))

    @pl.when(pl.program_id(0) == 0)
    def _init():
        acc_smem[0] = 0.0
        acc_smem[1] = 0.0
    acc_smem[0] += psum
    acc_smem[1] += pcnt

    @pl.when(pl.program_id(0) == N // NBLK - 1)
    def _fin():
        out_ref[...] = jnp.full((1, 1), acc_smem[0] / acc_smem[1], jnp.float32)


def _final(agg2, sel2d, wout2S, lin40, lin41, lin42,
           d1w, d1b, d2w, d2b, d3w, d3b):
    full = lambda shape: pl.BlockSpec(shape, lambda nb: (0,) * len(shape))
    return pl.pallas_call(
        _final_body,
        grid=(N // NBLK,),
        in_specs=[
            pl.BlockSpec((8, NC, NBLK, G), lambda nb: (0, 0, nb, 0)),
            pl.BlockSpec((NBLK, 1), lambda nb: (nb, 0)),
            full((15, C, C)),
            full((3 * C, C)),
            full((6 * C, C)),
            full((6 * C, C)),
            full((C, 250)),
            full((1, 250)),
            full((250, 150)),
            full((1, 150)),
            full((150, 1)),
            full((1, 1)),
        ],
        out_specs=pl.BlockSpec((1, 1), lambda nb: (0, 0)),
        out_shape=jax.ShapeDtypeStruct((1, 1), jnp.float32),
        scratch_shapes=[pltpu.SMEM((2,), jnp.float32)],
    )(agg2, sel2d, wout2S, lin40, lin41, lin42,
      d1w, d1b, d2w, d2b, d3w, d3b)


def _bd2(ws):
    z = jnp.zeros((24, G), jnp.float32)
    for j, w in enumerate(ws):
        z = z.at[12 * j:12 * (j + 1), 64 * j:64 * (j + 1)].set(w)
    return z


def _layer_stacks(ps, max_radius, n_basis, ng):
    mu = jnp.linspace(0.0, max_radius, n_basis)
    a = (1.0 / (mu[1] - mu[0])).reshape(1, 1)
    amu = (a[0, 0] * mu).reshape(1, n_basis)
    zw1 = jnp.zeros((n_basis, 12), jnp.float32)
    zb1 = jnp.zeros((12,), jnp.float32)
    zw2 = jnp.zeros((12, C), jnp.float32)
    zb2 = jnp.zeros((C,), jnp.float32)
    w1c = jnp.concatenate([p['w1'] if p is not None else zw1 for p in ps], 1)
    b1c = jnp.concatenate(
        [p['b1'] if p is not None else zb1 for p in ps]).reshape(1, -1)
    bdS = jnp.stack([
        _bd2([(ps[2 * g]['w2'] if ps[2 * g] is not None else zw2),
              (ps[2 * g + 1]['w2'] if ps[2 * g + 1] is not None else zw2)])
        for g in range(ng)])
    b2S = jnp.stack([
        jnp.concatenate(
            [(ps[2 * g]['b2'] if ps[2 * g] is not None else zb2),
             (ps[2 * g + 1]['b2'] if ps[2 * g + 1] is not None else zb2)])
        for g in range(ng)])[:, None, :]
    return a, amu, w1c, b1c, bdS, b2S


def kernel(x, edge_index, edge_attr, select_ca, params):
    src = edge_index[0].astype(jnp.int32)
    dst = edge_index[1].astype(jnp.int32)
    pad = EP - E
    padidx = (jnp.arange(pad, dtype=jnp.int32) * 997) % N
    src_p = jnp.concatenate([src, padidx])
    dst_p = jnp.concatenate([dst, padidx])
    idxpk = jnp.stack([src_p.reshape(EP // CH, CH),
                       dst_p.reshape(EP // CH, CH)], axis=1)
    ea_p = jnp.concatenate(
        [edge_attr, jnp.zeros((pad, 3), jnp.float32)], axis=0)
    zeros_tbl = jnp.zeros((ACC_N, G), jnp.float32)

    st1 = _layer_stacks(
        [params['conv1_%d' % l] for l in range(3)] + [None], 10.0, 20, 2)
    st2 = _layer_stacks(
        [params['conv2_%d%d%d' % c] for c in _COMBOS] + [None], 20.0, 40, 8)
    wins1 = [params['conv1_%d' % l]['win'] for l in range(3)]
    wincat1 = jnp.stack([
        jnp.concatenate([wins1[0], wins1[1]], axis=1),
        jnp.concatenate([wins1[2], jnp.zeros((C, C), jnp.float32)], axis=1),
    ])
    woutS1 = jnp.stack([params['conv1_%d' % l]['wout'] for l in range(3)])
    lin2S = jnp.stack([params['lin2_%d' % l] for l in range(3)])
    lin3S = jnp.stack([params['lin3_%d' % l] for l in range(3)])
    win2S = jnp.stack([params['conv2_%d%d%d' % c]['win'] for c in _COMBOS])
    wout2S = jnp.stack([params['conv2_%d%d%d' % c]['wout'] for c in _COMBOS])

    W1 = _edge_weights(2, 20, ea_p, *st1)
    W2 = _edge_weights(8, 40, ea_p, *st2)
    H1 = _node_h1(x, params['lin1'], wincat1)

    agg1 = _sc_edge_pass(2, H1.reshape(2 * N, G),
                         W1.reshape(2 * EP, G),
                         idxpk, zeros_tbl)
    H2 = _ec(agg1, woutS1, lin2S, lin3S, win2S)
    agg2 = _sc_edge_pass(8, H2.reshape(8 * N, G),
                         W2.reshape(8 * EP, G),
                         idxpk, zeros_tbl)

    sel2d = select_ca.reshape(N, 1).astype(jnp.int32)
    out = _final(agg2, sel2d, wout2S,
                 params['lin40'], params['lin41'], params['lin42'],
                 params['d1w'], params['d1b'].reshape(1, 250),
                 params['d2w'], params['d2b'].reshape(1, 150),
                 params['d3w'], params['d3b'].reshape(1, 1))
    return out[0, 0]
